# baseline (speedup 1.0000x reference)
import functools

import jax
import jax.numpy as jnp
import numpy as np
from jax import lax
from jax.experimental import pallas as pl
from jax.experimental.pallas import tpu as pltpu
from jax.experimental.pallas import tpu_sc as plsc

_NUM_NODES = 163842
_N_IN = 40962
_FEAT = 128
_HALF = 64
_HR_IN = 2 * _N_IN
_HR_OUT = 2 * _NUM_NODES
_G_TOTAL = _HR_OUT - _HR_IN

_NC, _NS = 2, 16
_NW = _NC * _NS
_GPW = _G_TOTAL // _NW
_CHUNKI = 128
_NCH = _GPW // _CHUNKI
_NBUF = 2

_CP_CHUNK = 512
_CP_PER_W = (_HR_IN // _NW) // _CP_CHUNK
_CP_REM = _HR_IN - _NW * _CP_PER_W * _CP_CHUNK

_AVG = np.zeros((_FEAT, _HALF), np.float32)
_AVG[2 * np.arange(_HALF), np.arange(_HALF)] = 0.5
_AVG[2 * np.arange(_HALF) + 1, np.arange(_HALF)] = 0.5

_OBLK = 8192
_NMM = pl.cdiv(_N_IN, _OBLK)
_NCP = pl.cdiv(_HR_IN, _OBLK)
_XH_OFF = _NMM * _OBLK
_SRC_ROWS = (_NMM + _NCP) * _OBLK


def _tc_body(x_mm_ref, x_cp_ref, m_ref, o_ref):
    pid = pl.program_id(0)

    @pl.when(pid < _NMM)
    def _():
        o_ref[...] = jnp.dot(x_mm_ref[...], m_ref[...],
                             preferred_element_type=jnp.float32)

    @pl.when(pid >= _NMM)
    def _():
        xc = x_cp_ref[...]
        o_ref[0::2] = xc[:, :_HALF]
        o_ref[1::2] = xc[:, _HALF:]


def _tc_stage(x):
    return pl.pallas_call(
        _tc_body,
        grid=(_NMM + _NCP,),
        in_specs=[
            pl.BlockSpec((_OBLK, _FEAT),
                         lambda i: (jnp.minimum(i, _NMM - 1), 0)),
            pl.BlockSpec((_OBLK // 2, _FEAT),
                         lambda i: (jnp.maximum(i - _NMM, 0), 0)),
            pl.BlockSpec((_FEAT, _HALF), lambda i: (0, 0)),
        ],
        out_specs=pl.BlockSpec((_OBLK, _HALF), lambda i: (i, 0)),
        out_shape=jax.ShapeDtypeStruct((_SRC_ROWS, _HALF), jnp.float32),
    )(x, x, jnp.asarray(_AVG))


def _sc_body(src_hbm, idx_hbm, out_hbm,
             idx_v, gbuf, cbuf, gsem0, gsem1):
    cid = lax.axis_index("c")
    sid = lax.axis_index("s")
    wid = sid * _NC + cid
    gsems = (gsem0, gsem1)

    base_cp = wid * _CP_PER_W * _CP_CHUNK
    for k in range(_CP_PER_W):
        off = base_cp + k * _CP_CHUNK
        pltpu.sync_copy(src_hbm.at[pl.ds(_XH_OFF + off, _CP_CHUNK)], cbuf)
        pltpu.sync_copy(cbuf, out_hbm.at[pl.ds(off, _CP_CHUNK)])

    @pl.when(wid == 0)
    def _():
        tail = _HR_IN - _CP_REM
        pltpu.sync_copy(src_hbm.at[pl.ds(_XH_OFF + tail, _CP_REM)],
                        cbuf.at[pl.ds(0, _CP_REM)])
        pltpu.sync_copy(cbuf.at[pl.ds(0, _CP_REM)],
                        out_hbm.at[pl.ds(tail, _CP_REM)])

    pltpu.sync_copy(idx_hbm.at[wid], idx_v)
    out_base = _HR_IN + wid * _GPW

    for b in range(_NBUF):
        pltpu.make_async_copy(
            src_hbm.at[idx_v.at[b]], gbuf.at[b], gsems[b]).start()

    def outer(g, carry):
        for b in range(_NBUF):
            c = g * _NBUF + b
            pltpu.make_async_copy(
                src_hbm.at[idx_v.at[c]], gbuf.at[b], gsems[b]).wait()
            pltpu.sync_copy(
                gbuf.at[b],
                out_hbm.at[pl.ds(out_base + c * _CHUNKI, _CHUNKI)])
            nxt = c + _NBUF

            @pl.when(nxt < _NCH)
            def _():
                pltpu.make_async_copy(
                    src_hbm.at[idx_v.at[nxt]], gbuf.at[b], gsems[b]).start()
        return carry

    lax.fori_loop(0, _NCH // _NBUF, outer, 0)


_sc_call = functools.partial(
    pl.kernel,
    out_type=jax.ShapeDtypeStruct((_HR_OUT, _HALF), jnp.float32),
    mesh=plsc.VectorSubcoreMesh(
        core_axis_name="c", subcore_axis_name="s",
        num_cores=_NC, num_subcores=_NS),
    scratch_types=[
        pltpu.VMEM((_NCH, _CHUNKI), jnp.int32),
        pltpu.VMEM((_NBUF, _CHUNKI, _HALF), jnp.float32),
        pltpu.VMEM((_CP_CHUNK, _HALF), jnp.float32),
        pltpu.SemaphoreType.DMA,
        pltpu.SemaphoreType.DMA,
    ],
    compiler_params=pltpu.CompilerParams(use_tc_tiling_on_sc=False),
)(_sc_body)


@jax.jit
def kernel(x, upsample_neighs_order):
    order = upsample_neighs_order.astype(jnp.int32)
    src = _tc_stage(x)
    idx3 = order.reshape(_NW, _NCH, _CHUNKI)
    out = _sc_call(src, idx3)
    return out.reshape(_NUM_NODES, _FEAT)


# speedup vs baseline: 1.6629x; 1.6629x over previous
import functools

import jax
import jax.numpy as jnp
import numpy as np
from jax import lax
from jax.experimental import pallas as pl
from jax.experimental.pallas import tpu as pltpu
from jax.experimental.pallas import tpu_sc as plsc

_NUM_NODES = 163842
_N_IN = 40962
_FEAT = 128
_HALF = 64
_HR_IN = 2 * _N_IN
_HR_OUT = 2 * _NUM_NODES
_G_TOTAL = _HR_OUT - _HR_IN

_NC, _NS = 2, 16
_NW = _NC * _NS
_GPW = _G_TOTAL // _NW
_CHUNKI = 128
_NCH = _GPW // _CHUNKI
_NBUF = 4

_CP_CHUNK = 256
_CP_PER_W = (_HR_IN // _NW) // _CP_CHUNK
_CP_REM = _HR_IN - _NW * _CP_PER_W * _CP_CHUNK

_AVG = np.zeros((_FEAT, _HALF), np.float32)
_AVG[2 * np.arange(_HALF), np.arange(_HALF)] = 0.5
_AVG[2 * np.arange(_HALF) + 1, np.arange(_HALF)] = 0.5

_ROWS_BLK = 8192


def _pair_avg_body(x_ref, m_ref, o_ref):
    o_ref[...] = jnp.dot(x_ref[...], m_ref[...],
                         preferred_element_type=jnp.float32)


def _pair_avg(x):
    grid = pl.cdiv(_N_IN, _ROWS_BLK)
    return pl.pallas_call(
        _pair_avg_body,
        grid=(grid,),
        in_specs=[
            pl.BlockSpec((_ROWS_BLK, _FEAT), lambda i: (i, 0)),
            pl.BlockSpec((_FEAT, _HALF), lambda i: (0, 0)),
        ],
        out_specs=pl.BlockSpec((_ROWS_BLK, _HALF), lambda i: (i, 0)),
        out_shape=jax.ShapeDtypeStruct((_N_IN, _HALF), jnp.float32),
    )(x, jnp.asarray(_AVG))


def _sc_body(xh_hbm, xp_hbm, idx_hbm, out_hbm,
             idx_v, gbuf, cbuf, grsem, gwsem, crsem, cwsem):
    cid = lax.axis_index("c")
    sid = lax.axis_index("s")
    wid = sid * _NC + cid

    def cp_read(k, b):
        off = base_cp + k * _CP_CHUNK
        return pltpu.make_async_copy(
            xh_hbm.at[pl.ds(off, _CP_CHUNK)], cbuf.at[b], crsem.at[b])

    def cp_write(k, b):
        off = base_cp + k * _CP_CHUNK
        return pltpu.make_async_copy(
            cbuf.at[b], out_hbm.at[pl.ds(off, _CP_CHUNK)], cwsem.at[b])

    base_cp = wid * _CP_PER_W * _CP_CHUNK
    cp_read(0, 0).start()
    cp_read(1, 1).start()
    for k in range(_CP_PER_W):
        b = k & 1
        cp_read(k, b).wait()
        cp_write(k, b).start()
        if k + 2 < _CP_PER_W:
            cp_write(k, b).wait()
            cp_read(k + 2, b).start()
    cp_write(_CP_PER_W - 2, _CP_PER_W & 1).wait()
    cp_write(_CP_PER_W - 1, (_CP_PER_W - 1) & 1).wait()

    @pl.when(wid == 0)
    def _():
        tail = _HR_IN - _CP_REM
        pltpu.sync_copy(xh_hbm.at[pl.ds(tail, _CP_REM)],
                        cbuf.at[0].at[pl.ds(0, _CP_REM)])
        pltpu.sync_copy(cbuf.at[0].at[pl.ds(0, _CP_REM)],
                        out_hbm.at[pl.ds(tail, _CP_REM)])

    pltpu.sync_copy(idx_hbm.at[wid], idx_v)
    out_base = _HR_IN + wid * _GPW

    def g_read(c, b):
        return pltpu.make_async_copy(
            xp_hbm.at[idx_v.at[c]], gbuf.at[b], grsem.at[b])

    def g_write(c, b):
        return pltpu.make_async_copy(
            gbuf.at[b],
            out_hbm.at[pl.ds(out_base + c * _CHUNKI, _CHUNKI)],
            gwsem.at[b])

    for b in range(_NBUF):
        g_read(b, b).start()

    def outer(g, carry):
        for b in range(_NBUF):
            c = g * _NBUF + b
            g_read(c, b).wait()
            g_write(c, b).start()
            nxt = c + _NBUF

            @pl.when(nxt < _NCH)
            def _():
                g_write(c, b).wait()
                g_read(nxt, b).start()
        return carry

    lax.fori_loop(0, _NCH // _NBUF, outer, 0)

    for b in range(_NBUF):
        g_write(_NCH - _NBUF + b, b).wait()


_sc_call = functools.partial(
    pl.kernel,
    out_type=jax.ShapeDtypeStruct((_HR_OUT, _HALF), jnp.float32),
    mesh=plsc.VectorSubcoreMesh(
        core_axis_name="c", subcore_axis_name="s",
        num_cores=_NC, num_subcores=_NS),
    scratch_types=[
        pltpu.VMEM((_NCH, _CHUNKI), jnp.int32),
        pltpu.VMEM((_NBUF, _CHUNKI, _HALF), jnp.float32),
        pltpu.VMEM((2, _CP_CHUNK, _HALF), jnp.float32),
        pltpu.SemaphoreType.DMA((_NBUF,)),
        pltpu.SemaphoreType.DMA((_NBUF,)),
        pltpu.SemaphoreType.DMA((2,)),
        pltpu.SemaphoreType.DMA((2,)),
    ],
    compiler_params=pltpu.CompilerParams(use_tc_tiling_on_sc=False),
)(_sc_body)


@jax.jit
def kernel(x, upsample_neighs_order):
    order = upsample_neighs_order.astype(jnp.int32)
    xp = _pair_avg(x)
    xh = x.reshape(_HR_IN, _HALF)
    idx3 = order.reshape(_NW, _NCH, _CHUNKI)
    out = _sc_call(xh, xp, idx3)
    return out.reshape(_NUM_NODES, _FEAT)
